# baseline (speedup 1.0000x reference)
import functools

import jax
import jax.numpy as jnp
from jax import lax
from jax.experimental import pallas as pl
from jax.experimental.pallas import tpu as pltpu
from jax.experimental.pallas import tpu_sc as plsc

VOCAB = 100000
D = 128
B = 4096
L = 200
TOT = B * L
NC = 2
NS = 16
NW = NC * NS
PER_W = TOT // NW
CHUNK = 128
NCH = PER_W // CHUNK
RING = 4
LOOKAHEAD = 2

_mesh = plsc.VectorSubcoreMesh(core_axis_name="c", subcore_axis_name="s")


@functools.partial(
    pl.kernel,
    mesh=_mesh,
    out_type=jax.ShapeDtypeStruct((TOT, D), jnp.float32),
    scratch_types=[
        pltpu.VMEM((NCH, CHUNK), jnp.int32),
        pltpu.VMEM((RING, CHUNK, D), jnp.float32),
    ] + [pltpu.SemaphoreType.DMA] * (2 * RING),
)
def _embed_gather(idx_hbm, table_hbm, out_hbm, idx_v, stage, *sems):
    gsems, ssems = sems[:RING], sems[RING:]
    cid = lax.axis_index("c")
    sid = lax.axis_index("s")
    wid = sid * NC + cid
    pltpu.sync_copy(idx_hbm.at[wid], idx_v)
    base = wid * PER_W

    def out_slab(j):
        return out_hbm.at[pl.ds(base + j * CHUNK, CHUNK)]

    def start_gather(j, r):
        pltpu.async_copy(table_hbm.at[idx_v.at[j]], stage.at[r], gsems[r])

    def wait_gather(r):
        pltpu.make_async_copy(
            table_hbm.at[pl.ds(0, CHUNK)], stage.at[r], gsems[r]
        ).wait()

    for r in range(LOOKAHEAD):
        start_gather(r, r)

    def round_body(i, carry):
        j0 = i * RING
        for r in range(RING):
            j = j0 + r
            wait_gather(r)
            pltpu.async_copy(stage.at[r], out_slab(j), ssems[r])
            jn = j + LOOKAHEAD
            rn = (r + LOOKAHEAD) % RING

            @pl.when(jn < NCH)
            def _():
                @pl.when(jn >= RING)
                def _():
                    pltpu.make_async_copy(
                        stage.at[rn], out_slab(jn - RING), ssems[rn]
                    ).wait()

                start_gather(jn, rn)
        return carry

    lax.fori_loop(0, NCH // RING, round_body, 0)

    for r in range(RING):
        j = NCH - RING + r
        pltpu.make_async_copy(stage.at[r], out_slab(j), ssems[r]).wait()


def kernel(w_id, table):
    idx = w_id.astype(jnp.int32).reshape(NW, NCH, CHUNK)
    out = _embed_gather(idx, table)
    return out.reshape(B, L, D)


# speedup vs baseline: 1.0021x; 1.0021x over previous
import functools

import jax
import jax.numpy as jnp
from jax import lax
from jax.experimental import pallas as pl
from jax.experimental.pallas import tpu as pltpu
from jax.experimental.pallas import tpu_sc as plsc

VOCAB = 100000
D = 128
B = 4096
L = 200
TOT = B * L
NC = 2
NS = 16
NW = NC * NS
PER_W = TOT // NW
CHUNK = 128
NCH = PER_W // CHUNK
RING = 4
LOOKAHEAD = 3

_mesh = plsc.VectorSubcoreMesh(core_axis_name="c", subcore_axis_name="s")


@functools.partial(
    pl.kernel,
    mesh=_mesh,
    out_type=jax.ShapeDtypeStruct((TOT, D), jnp.float32),
    scratch_types=[
        pltpu.VMEM((NCH, CHUNK), jnp.int32),
        pltpu.VMEM((RING, CHUNK, D), jnp.float32),
    ] + [pltpu.SemaphoreType.DMA] * (2 * RING),
)
def _embed_gather(idx_hbm, table_hbm, out_hbm, idx_v, stage, *sems):
    gsems, ssems = sems[:RING], sems[RING:]
    cid = lax.axis_index("c")
    sid = lax.axis_index("s")
    wid = sid * NC + cid
    pltpu.sync_copy(idx_hbm.at[wid], idx_v)
    base = wid * PER_W

    def out_slab(j):
        return out_hbm.at[pl.ds(base + j * CHUNK, CHUNK)]

    def start_gather(j, r):
        pltpu.async_copy(table_hbm.at[idx_v.at[j]], stage.at[r], gsems[r])

    def wait_gather(r):
        pltpu.make_async_copy(
            table_hbm.at[pl.ds(0, CHUNK)], stage.at[r], gsems[r]
        ).wait()

    for r in range(LOOKAHEAD):
        start_gather(r, r)

    def round_body(i, carry):
        j0 = i * RING
        for r in range(RING):
            j = j0 + r
            wait_gather(r)
            pltpu.async_copy(stage.at[r], out_slab(j), ssems[r])
            jn = j + LOOKAHEAD
            rn = (r + LOOKAHEAD) % RING

            @pl.when(jn < NCH)
            def _():
                @pl.when(jn >= RING)
                def _():
                    pltpu.make_async_copy(
                        stage.at[rn], out_slab(jn - RING), ssems[rn]
                    ).wait()

                start_gather(jn, rn)
        return carry

    lax.fori_loop(0, NCH // RING, round_body, 0)

    for r in range(RING):
        j = NCH - RING + r
        pltpu.make_async_copy(stage.at[r], out_slab(j), ssems[r]).wait()


def kernel(w_id, table):
    idx = w_id.astype(jnp.int32).reshape(NW, NCH, CHUNK)
    out = _embed_gather(idx, table)
    return out.reshape(B, L, D)


# speedup vs baseline: 1.0064x; 1.0042x over previous
import functools

import jax
import jax.numpy as jnp
from jax import lax
from jax.experimental import pallas as pl
from jax.experimental.pallas import tpu as pltpu
from jax.experimental.pallas import tpu_sc as plsc

VOCAB = 100000
D = 128
B = 4096
L = 200
TOT = B * L
NC = 2
NS = 16
NW = NC * NS
PER_W = TOT // NW
CHUNK = 128
NCH = PER_W // CHUNK
RING = 5
LOOKAHEAD = 3

_mesh = plsc.VectorSubcoreMesh(core_axis_name="c", subcore_axis_name="s")


@functools.partial(
    pl.kernel,
    mesh=_mesh,
    out_type=jax.ShapeDtypeStruct((TOT, D), jnp.float32),
    scratch_types=[
        pltpu.VMEM((NCH, CHUNK), jnp.int32),
        pltpu.VMEM((RING, CHUNK, D), jnp.float32),
    ] + [pltpu.SemaphoreType.DMA] * (2 * RING),
)
def _embed_gather(idx_hbm, table_hbm, out_hbm, idx_v, stage, *sems):
    gsems, ssems = sems[:RING], sems[RING:]
    cid = lax.axis_index("c")
    sid = lax.axis_index("s")
    wid = sid * NC + cid
    pltpu.sync_copy(idx_hbm.at[wid], idx_v)
    base = wid * PER_W

    def out_slab(j):
        return out_hbm.at[pl.ds(base + j * CHUNK, CHUNK)]

    def start_gather(j, r):
        pltpu.async_copy(table_hbm.at[idx_v.at[j]], stage.at[r], gsems[r])

    def wait_gather(r):
        pltpu.make_async_copy(
            table_hbm.at[pl.ds(0, CHUNK)], stage.at[r], gsems[r]
        ).wait()

    for r in range(LOOKAHEAD):
        start_gather(r, r)

    def round_body(i, carry):
        j0 = i * RING
        for r in range(RING):
            j = j0 + r
            wait_gather(r)
            pltpu.async_copy(stage.at[r], out_slab(j), ssems[r])
            jn = j + LOOKAHEAD
            rn = (r + LOOKAHEAD) % RING

            @pl.when(jn < NCH)
            def _():
                @pl.when(jn >= RING)
                def _():
                    pltpu.make_async_copy(
                        stage.at[rn], out_slab(jn - RING), ssems[rn]
                    ).wait()

                start_gather(jn, rn)
        return carry

    lax.fori_loop(0, NCH // RING, round_body, 0)

    for r in range(RING):
        j = NCH - RING + r
        pltpu.make_async_copy(stage.at[r], out_slab(j), ssems[r]).wait()


def kernel(w_id, table):
    idx = w_id.astype(jnp.int32).reshape(NW, NCH, CHUNK)
    out = _embed_gather(idx, table)
    return out.reshape(B, L, D)
